# baseline (speedup 1.0000x reference)
import functools

import jax
import jax.numpy as jnp
from jax import lax
from jax.experimental import pallas as pl
from jax.experimental.pallas import tpu as pltpu
from jax.experimental.pallas import tpu_sc as plsc

B = 4096
L = 20
D = 1000
NW = 32
BW = B // NW
DC = 8
ND = D // DC
NBUF = 2
TPW = L * ND


def _sc_one_hot(xt_flat):
    mesh = plsc.VectorSubcoreMesh(core_axis_name="c", subcore_axis_name="s")

    @functools.partial(
        pl.kernel,
        mesh=mesh,
        compiler_params=pltpu.CompilerParams(needs_layout_passes=False),
        out_type=jax.ShapeDtypeStruct((L, D, B), jnp.float32),
        scratch_types=[
            pltpu.VMEM((L * BW,), jnp.int32),
            pltpu.VMEM((DC, BW), jnp.float32),
            pltpu.VMEM((DC, BW), jnp.float32),
            pltpu.SemaphoreType.DMA,
            pltpu.SemaphoreType.DMA,
        ],
    )
    def one_hot_kernel(xt_hbm, out_hbm, xv, buf0, buf1, sem0, sem1):
        wid = lax.axis_index("s") * 2 + lax.axis_index("c")
        b0 = wid * BW
        bufs = (buf0, buf1)
        sems = (sem0, sem1)

        lanes = lax.iota(jnp.int32, 16)
        ones_v = jnp.full((16,), 1.0, jnp.float32)
        zeros_v = jnp.zeros((16,), jnp.float32)

        def stage(l):
            return pltpu.make_async_copy(
                xt_hbm.at[pl.ds(l * B + b0, BW)], xv.at[pl.ds(l * BW, BW)], sem0
            )

        for l in range(L):
            stage(l).start()

        def zero_body(d, carry):
            for j in range(BW // 16):
                buf0[d, pl.ds(j * 16, 16)] = zeros_v
                buf1[d, pl.ds(j * 16, 16)] = zeros_v
            return carry

        lax.fori_loop(0, DC, zero_body, 0)
        for l in range(L):
            stage(l).wait()

        def scat(buf, t, val_v):
            l = t // ND
            d0 = (t % ND) * DC
            for j in range(BW // 16):
                d_idx = xv[pl.ds(l * BW + j * 16, 16)] - d0
                m = (d_idx >= 0) & (d_idx < DC)
                plsc.store_scatter(buf, [d_idx, lanes + j * 16], val_v, mask=m)

        def dma(b, t):
            l = t // ND
            d0 = (t % ND) * DC
            return pltpu.make_async_copy(
                bufs[b],
                out_hbm.at[l, pl.ds(d0, DC), pl.ds(b0, BW)],
                sems[b],
            )

        for b in range(NBUF):
            scat(bufs[b], b, ones_v)
            dma(b, b).start()

        def step(g0, carry):
            for b in range(NBUF):
                t = g0 * NBUF + b
                dma(b, t - NBUF).wait()
                scat(bufs[b], t - NBUF, zeros_v)
                scat(bufs[b], t, ones_v)
                dma(b, t).start()
            return carry

        lax.fori_loop(1, TPW // NBUF, step, 0)

        for b in range(NBUF):
            dma(b, TPW - NBUF + b).wait()

    return one_hot_kernel(xt_flat)


def kernel(X_in, ones):
    del ones
    xt = X_in.astype(jnp.int32).T.reshape(-1)
    y = _sc_one_hot(xt)
    return jnp.transpose(y, (2, 1, 0))


# speedup vs baseline: 2.7848x; 2.7848x over previous
import functools

import jax
import jax.numpy as jnp
from jax import lax
from jax.experimental import pallas as pl
from jax.experimental.pallas import tpu as pltpu
from jax.experimental.pallas import tpu_sc as plsc

B = 4096
L = 20
D = 1000
NW = 32
BW = B // NW
DC = 40
ND = D // DC
NBUF = 4
TPW = L * ND


def _sc_one_hot(xt_flat):
    mesh = plsc.VectorSubcoreMesh(core_axis_name="c", subcore_axis_name="s")

    @functools.partial(
        pl.kernel,
        mesh=mesh,
        compiler_params=pltpu.CompilerParams(needs_layout_passes=False),
        out_type=jax.ShapeDtypeStruct((L, D, B), jnp.float32),
        scratch_types=[
            pltpu.VMEM((L * BW,), jnp.int32),
            pltpu.VMEM((DC, BW), jnp.float32),
            pltpu.VMEM((DC, BW), jnp.float32),
            pltpu.VMEM((DC, BW), jnp.float32),
            pltpu.VMEM((DC, BW), jnp.float32),
            pltpu.SemaphoreType.DMA,
            pltpu.SemaphoreType.DMA,
            pltpu.SemaphoreType.DMA,
            pltpu.SemaphoreType.DMA,
        ],
    )
    def one_hot_kernel(xt_hbm, out_hbm, xv, buf0, buf1, buf2, buf3, sem0, sem1, sem2, sem3):
        wid = lax.axis_index("s") * 2 + lax.axis_index("c")
        b0 = wid * BW
        bufs = (buf0, buf1, buf2, buf3)
        sems = (sem0, sem1, sem2, sem3)

        lanes = lax.iota(jnp.int32, 16)
        ones_v = jnp.full((16,), 1.0, jnp.float32)
        zeros_v = jnp.zeros((16,), jnp.float32)

        def stage(l):
            return pltpu.make_async_copy(
                xt_hbm.at[pl.ds(l * B + b0, BW)], xv.at[pl.ds(l * BW, BW)], sem0
            )

        for l in range(L):
            stage(l).start()

        def zero_body(d, carry):
            for j in range(BW // 16):
                for bb in bufs:
                    bb[d, pl.ds(j * 16, 16)] = zeros_v
            return carry

        lax.fori_loop(0, DC, zero_body, 0)
        for l in range(L):
            stage(l).wait()

        def scat(buf, t, val_v):
            l = t // ND
            d0 = (t % ND) * DC
            for j in range(BW // 16):
                d_idx = xv[pl.ds(l * BW + j * 16, 16)] - d0
                m = (d_idx >= 0) & (d_idx < DC)
                plsc.store_scatter(buf, [d_idx, lanes + j * 16], val_v, mask=m)

        def dma(b, t):
            l = t // ND
            d0 = (t % ND) * DC
            return pltpu.make_async_copy(
                bufs[b],
                out_hbm.at[l, pl.ds(d0, DC), pl.ds(b0, BW)],
                sems[b],
            )

        for b in range(NBUF):
            scat(bufs[b], b, ones_v)
            dma(b, b).start()

        def step(g0, carry):
            for b in range(NBUF):
                t = g0 * NBUF + b
                dma(b, t - NBUF).wait()
                scat(bufs[b], t - NBUF, zeros_v)
                scat(bufs[b], t, ones_v)
                dma(b, t).start()
            return carry

        lax.fori_loop(1, TPW // NBUF, step, 0)

        for b in range(NBUF):
            dma(b, TPW - NBUF + b).wait()

    return one_hot_kernel(xt_flat)


def kernel(X_in, ones):
    del ones
    xt = X_in.astype(jnp.int32).T.reshape(-1)
    y = _sc_one_hot(xt)
    return jnp.transpose(y, (2, 1, 0))


# speedup vs baseline: 2.8329x; 1.0173x over previous
import functools

import jax
import jax.numpy as jnp
from jax import lax
from jax.experimental import pallas as pl
from jax.experimental.pallas import tpu as pltpu
from jax.experimental.pallas import tpu_sc as plsc

B = 4096
L = 20
D = 1000
NW = 32
BW = B // NW
DC = 40
ND = D // DC
NBUF = 2
TPW = L * ND


def _sc_one_hot(xt_flat):
    mesh = plsc.VectorSubcoreMesh(core_axis_name="c", subcore_axis_name="s")

    @functools.partial(
        pl.kernel,
        mesh=mesh,
        compiler_params=pltpu.CompilerParams(needs_layout_passes=False),
        out_type=jax.ShapeDtypeStruct((L, D, B), jnp.float32),
        scratch_types=[
            pltpu.VMEM((L * BW,), jnp.int32),
            pltpu.VMEM((DC, BW), jnp.float32),
            pltpu.VMEM((DC, BW), jnp.float32),
            pltpu.SemaphoreType.DMA,
            pltpu.SemaphoreType.DMA,
        ],
    )
    def one_hot_kernel(xt_hbm, out_hbm, xv, buf0, buf1, sem0, sem1):
        wid = lax.axis_index("s") * 2 + lax.axis_index("c")
        b0 = wid * BW
        bufs = (buf0, buf1)
        sems = (sem0, sem1)

        lanes = lax.iota(jnp.int32, 16)
        ones_v = jnp.full((16,), 1.0, jnp.float32)
        zeros_v = jnp.zeros((16,), jnp.float32)

        def stage(l):
            return pltpu.make_async_copy(
                xt_hbm.at[pl.ds(l * B + b0, BW)], xv.at[pl.ds(l * BW, BW)], sem0
            )

        for l in range(L):
            stage(l).start()

        def zero_body(d, carry):
            for j in range(BW // 16):
                buf0[d, pl.ds(j * 16, 16)] = zeros_v
                buf1[d, pl.ds(j * 16, 16)] = zeros_v
            return carry

        lax.fori_loop(0, DC, zero_body, 0)
        for l in range(L):
            stage(l).wait()

        def scat(buf, t, val_v):
            l = t // ND
            d0 = (t % ND) * DC
            for j in range(BW // 16):
                d_idx = xv[pl.ds(l * BW + j * 16, 16)] - d0
                m = (d_idx >= 0) & (d_idx < DC)
                plsc.store_scatter(buf, [d_idx, lanes + j * 16], val_v, mask=m)

        def dma(b, t):
            l = t // ND
            d0 = (t % ND) * DC
            return pltpu.make_async_copy(
                bufs[b],
                out_hbm.at[l, pl.ds(d0, DC), pl.ds(b0, BW)],
                sems[b],
            )

        for b in range(NBUF):
            scat(bufs[b], b, ones_v)
            dma(b, b).start()

        def step(g0, carry):
            for b in range(NBUF):
                t = g0 * NBUF + b
                dma(b, t - NBUF).wait()
                scat(bufs[b], t - NBUF, zeros_v)
                scat(bufs[b], t, ones_v)
                dma(b, t).start()
            return carry

        lax.fori_loop(1, TPW // NBUF, step, 0)

        for b in range(NBUF):
            dma(b, TPW - NBUF + b).wait()

    return one_hot_kernel(xt_flat)


def kernel(X_in, ones):
    del ones
    xt = X_in.astype(jnp.int32).T.reshape(-1)
    y = _sc_one_hot(xt)
    return jnp.transpose(y, (2, 1, 0))
